# baseline (speedup 1.0000x reference)
import jax
import jax.numpy as jnp
from jax import lax
from jax.experimental import pallas as pl
from jax.experimental.pallas import tpu as pltpu
from jax.experimental.pallas import tpu_sc as plsc

VOCAB = 1000000
DIM = 64
BATCH = 16384

_info = plsc.get_sparse_core_info()
NC, NS, L = _info.num_cores, _info.num_subcores, _info.num_lanes
NW = NC * NS
BPW = BATCH // NW
CHUNK = 128

_PACK_W = 4096
_PACK_H = _PACK_W // 2
_NBLK = VOCAB // _PACK_W
_T = 100
_NB_SC = _NBLK - _T
_SLAB = 256
_PAIRS_PW = _NB_SC * (_PACK_H // _SLAB) // NW
_TC_ROWS = (_T + 1) * _PACK_H
_SC_ROWS = _NB_SC * _PACK_H


def _pack_body(wt_ref, ct_ref, ow_ref, oc_ref):
    w = wt_ref[...]
    ow_ref[...] = jnp.concatenate(
        [w[:, :_PACK_H].T, w[:, _PACK_H:].T], axis=1)
    c = ct_ref[...]
    oc_ref[...] = jnp.concatenate(
        [c[:, :_PACK_H].T, c[:, _PACK_H:].T], axis=1)


def _tc_pack(wt, ct):
    def in_map(g):
        return (0, jnp.where(g < _T, g, _NBLK))

    return pl.pallas_call(
        _pack_body,
        grid=(_T + 1,),
        in_specs=[
            pl.BlockSpec((DIM, _PACK_W), in_map),
            pl.BlockSpec((DIM, _PACK_W), in_map),
        ],
        out_specs=[
            pl.BlockSpec((_PACK_H, 128), lambda g: (g, 0)),
            pl.BlockSpec((_PACK_H, 128), lambda g: (g, 0)),
        ],
        out_shape=(
            jax.ShapeDtypeStruct((_TC_ROWS, 128), jnp.float32),
            jax.ShapeDtypeStruct((_TC_ROWS, 128), jnp.float32),
        ),
    )(wt, ct)


def _sc_pack_body(wt_hbm, ct_hbm, wp_hbm, cp_hbm,
                  slab_a, slab_b, out_b, sem_a, sem_b):
    wid = lax.axis_index("s") * NC + lax.axis_index("c")
    dvecs = [lax.iota(jnp.int32, L) + q * L for q in range(DIM // L)]

    for src, dst in ((wt_hbm, wp_hbm), (ct_hbm, cp_hbm)):
        def pair_step(p, _, src=src, dst=dst):
            ap = wid * _PAIRS_PW + p
            g = _T + (ap >> 3)
            s0 = (ap & 7) * _SLAB
            ca = g * _PACK_W + s0
            cp_a = pltpu.async_copy(src.at[:, pl.ds(ca, _SLAB)], slab_a,
                                    sem_a)
            cp_b = pltpu.async_copy(src.at[:, pl.ds(ca + _PACK_H, _SLAB)],
                                    slab_b, sem_b)
            cp_a.wait()
            cp_b.wait()

            def row_step(s, _):
                sv = jnp.full((L,), s, jnp.int32)
                for q in range(DIM // L):
                    out_b[s, pl.ds(q * L, L)] = plsc.load_gather(
                        slab_a, [dvecs[q], sv])
                    out_b[s, pl.ds(DIM + q * L, L)] = plsc.load_gather(
                        slab_b, [dvecs[q], sv])
                return 0

            lax.fori_loop(0, _SLAB, row_step, 0)
            r0 = (g - _T) * _PACK_H + s0
            pltpu.sync_copy(out_b, dst.at[pl.ds(r0, _SLAB)])
            return 0

        lax.fori_loop(0, _PAIRS_PW, pair_step, 0)


def _sc_pack(wt, ct):
    mesh = plsc.VectorSubcoreMesh(core_axis_name="c", subcore_axis_name="s")
    f = pl.kernel(
        _sc_pack_body,
        out_type=(
            jax.ShapeDtypeStruct((_SC_ROWS, 128), jnp.float32),
            jax.ShapeDtypeStruct((_SC_ROWS, 128), jnp.float32),
        ),
        mesh=mesh,
        compiler_params=pltpu.CompilerParams(needs_layout_passes=False),
        scratch_types=[
            pltpu.VMEM((DIM, _SLAB), jnp.float32),
            pltpu.VMEM((DIM, _SLAB), jnp.float32),
            pltpu.VMEM((_SLAB, 128), jnp.float32),
            pltpu.SemaphoreType.DMA,
            pltpu.SemaphoreType.DMA,
        ],
    )
    return f(wt, ct)


def _dot_body(rt_i_hbm, rs_i_hbm, mt_i_hbm, rt_j_hbm, rs_j_hbm, mt_j_hbm,
              wp_tc_hbm, wp_sc_hbm, cp_tc_hbm, cp_sc_hbm, partials_hbm,
              rt_i_v, rs_i_v, mt_i_v, rt_j_v, rs_j_v, mt_j_v,
              buf_a, buf_b, buf_c, buf_d, acc_v,
              sem_a, sem_b, sem_c, sem_d):
    wid = lax.axis_index("s") * NC + lax.axis_index("c")
    base = wid * BPW

    for hbm, v in ((rt_i_hbm, rt_i_v), (rs_i_hbm, rs_i_v),
                   (mt_i_hbm, mt_i_v), (rt_j_hbm, rt_j_v),
                   (rs_j_hbm, rs_j_v), (mt_j_hbm, mt_j_v)):
        pltpu.sync_copy(hbm.at[pl.ds(base, BPW)], v)

    zero = jnp.zeros((L,), jnp.float32)
    accs = (zero, zero, zero, zero)
    for chunk in range(BPW // CHUNK):
        cb = chunk * CHUNK
        cps = [
            pltpu.async_copy(wp_tc_hbm.at[rt_i_v.at[pl.ds(cb, CHUNK)]],
                             buf_a, sem_a),
            pltpu.async_copy(wp_sc_hbm.at[rs_i_v.at[pl.ds(cb, CHUNK)]],
                             buf_b, sem_b),
            pltpu.async_copy(cp_tc_hbm.at[rt_j_v.at[pl.ds(cb, CHUNK)]],
                             buf_c, sem_c),
            pltpu.async_copy(cp_sc_hbm.at[rs_j_v.at[pl.ds(cb, CHUNK)]],
                             buf_d, sem_d),
        ]
        for cp in cps:
            cp.wait()

        def dot_group(g, accs):
            a0, a1, a2, a3 = accs
            mi = mt_i_v[pl.ds(cb + g * L, L)]
            mj = mt_j_v[pl.ds(cb + g * L, L)]
            for t in range(L):
                k = g * L + t
                oi = mi[t] & DIM
                oj = mj[t] & DIM
                si = mi[t] >> 7
                sj = mj[t] >> 7

                def w_at(c):
                    return jnp.where(
                        si == 1,
                        buf_a[k, pl.ds(oi + c * L, L)],
                        buf_b[k, pl.ds(oi + c * L, L)])

                def c_at(c):
                    return jnp.where(
                        sj == 1,
                        buf_c[k, pl.ds(oj + c * L, L)],
                        buf_d[k, pl.ds(oj + c * L, L)])

                a0 = a0 + w_at(0) * c_at(0)
                a1 = a1 + w_at(1) * c_at(1)
                a2 = a2 + w_at(2) * c_at(2)
                a3 = a3 + w_at(3) * c_at(3)
            return (a0, a1, a2, a3)

        accs = lax.fori_loop(0, CHUNK // L, dot_group, accs)

    a0, a1, a2, a3 = accs
    acc_v[pl.ds(0, L)] = a0
    acc_v[pl.ds(L, L)] = a1
    acc_v[pl.ds(2 * L, L)] = a2
    acc_v[pl.ds(3 * L, L)] = a3
    zv = jnp.zeros((L,), jnp.float32)
    for z in range(4, 8):
        acc_v[pl.ds(z * L, L)] = zv
    pltpu.sync_copy(acc_v, partials_hbm.at[pl.ds(wid * 128, 128)])


def _sc_dot(rt_i, rs_i, mt_i, rt_j, rs_j, mt_j, wp_tc, wp_sc, cp_tc, cp_sc):
    mesh = plsc.VectorSubcoreMesh(core_axis_name="c", subcore_axis_name="s")
    f = pl.kernel(
        _dot_body,
        out_type=jax.ShapeDtypeStruct((NW * 128,), jnp.float32),
        mesh=mesh,
        scratch_types=[
            pltpu.VMEM((BPW,), jnp.int32),
            pltpu.VMEM((BPW,), jnp.int32),
            pltpu.VMEM((BPW,), jnp.int32),
            pltpu.VMEM((BPW,), jnp.int32),
            pltpu.VMEM((BPW,), jnp.int32),
            pltpu.VMEM((BPW,), jnp.int32),
            pltpu.VMEM((CHUNK, 128), jnp.float32),
            pltpu.VMEM((CHUNK, 128), jnp.float32),
            pltpu.VMEM((CHUNK, 128), jnp.float32),
            pltpu.VMEM((CHUNK, 128), jnp.float32),
            pltpu.VMEM((128,), jnp.float32),
            pltpu.SemaphoreType.DMA,
            pltpu.SemaphoreType.DMA,
            pltpu.SemaphoreType.DMA,
            pltpu.SemaphoreType.DMA,
        ],
    )
    return f(rt_i, rs_i, mt_i, rt_j, rs_j, mt_j, wp_tc, wp_sc, cp_tc, cp_sc)


def _bias_body(w_i_hbm, w_j_hbm, w_bias_hbm, c_bias_hbm, bias_hbm,
               idx_i_v, idx_j_v, bi_v, bj_v, sem_bi, sem_bj):
    wid = lax.axis_index("s") * NC + lax.axis_index("c")
    base = wid * BPW

    pltpu.sync_copy(w_i_hbm.at[pl.ds(base, BPW)], idx_i_v)
    pltpu.sync_copy(w_j_hbm.at[pl.ds(base, BPW)], idx_j_v)

    cp_bi = pltpu.async_copy(w_bias_hbm.at[idx_i_v], bi_v, sem_bi)
    cp_bj = pltpu.async_copy(c_bias_hbm.at[idx_j_v], bj_v, sem_bj)
    cp_bi.wait()
    cp_bj.wait()

    def bias_step(k, _):
        s = pl.ds(k * L, L)
        bi_v[s] = bi_v[s] + bj_v[s]
        return 0

    lax.fori_loop(0, BPW // L, bias_step, 0, unroll=4)
    pltpu.sync_copy(bi_v, bias_hbm.at[pl.ds(base, BPW)])


def _sc_bias(w_i, w_j, w_bias, c_bias):
    mesh = plsc.VectorSubcoreMesh(core_axis_name="c", subcore_axis_name="s")
    f = pl.kernel(
        _bias_body,
        out_type=jax.ShapeDtypeStruct((BATCH,), jnp.float32),
        mesh=mesh,
        compiler_params=pltpu.CompilerParams(use_tc_tiling_on_sc=False),
        scratch_types=[
            pltpu.VMEM((BPW,), jnp.int32),
            pltpu.VMEM((BPW,), jnp.int32),
            pltpu.VMEM((BPW,), jnp.float32),
            pltpu.VMEM((BPW,), jnp.float32),
            pltpu.SemaphoreType.DMA,
            pltpu.SemaphoreType.DMA,
        ],
    )
    return f(w_i, w_j, w_bias, c_bias)


def _tc_body(partials_ref, bias_ref, x_ref, loss_ref):
    x = jnp.sum(partials_ref[...])
    b = bias_ref[...]
    y_true = jnp.abs(b) + 1e-6
    t = jnp.abs(x) / 100.0 + jnp.zeros_like(b)
    weight = jnp.exp(0.75 * jnp.log(t))
    loss_ref[...] = weight * jnp.square(x - jnp.log(y_true))
    x_ref[...] = jnp.broadcast_to(x, (1, 1))


def _tc_loss(partials, bias2d):
    return pl.pallas_call(
        _tc_body,
        out_shape=(
            jax.ShapeDtypeStruct((1, 1), jnp.float32),
            jax.ShapeDtypeStruct(bias2d.shape, jnp.float32),
        ),
    )(partials, bias2d)


def kernel(w_i, w_j, w_emb, c_emb, w_bias, c_bias):
    w_i = w_i.astype(jnp.int32)
    w_j = w_j.astype(jnp.int32)

    def remap(v):
        g = v >> 12
        in_tc = (g < _T) | (g == _NBLK)
        row_local = (v & 2047)
        row_tc = jnp.where(g == _NBLK, _T * _PACK_H + row_local,
                           (g << 11) + row_local)
        row_sc = ((g - _T) << 11) + row_local
        rt = jnp.where(in_tc, row_tc, 0).astype(jnp.int32)
        rs = jnp.where(in_tc, 0, row_sc).astype(jnp.int32)
        off = ((v >> 11) & 1) << 6
        meta = (off | jnp.where(in_tc, 128, 0)).astype(jnp.int32)
        return rt, rs, meta

    rt_i, rs_i, mt_i = remap(w_i)
    rt_j, rs_j, mt_j = remap(w_j)

    wp_tc, cp_tc = _tc_pack(w_emb.T, c_emb.T)
    wp_sc, cp_sc = _sc_pack(w_emb.T, c_emb.T)
    partials = _sc_dot(rt_i, rs_i, mt_i, rt_j, rs_j, mt_j,
                       wp_tc, wp_sc, cp_tc, cp_sc)
    bias = _sc_bias(w_i, w_j, w_bias, c_bias)
    x, loss = _tc_loss(partials.reshape(NW, 128), bias.reshape(128, 128))
    return (x.reshape(()), loss.reshape(BATCH))


# speedup vs baseline: 5.6398x; 5.6398x over previous
import jax
import jax.numpy as jnp
from jax import lax
from jax.experimental import pallas as pl
from jax.experimental.pallas import tpu as pltpu
from jax.experimental.pallas import tpu_sc as plsc

VOCAB = 1000000
DIM = 64
BATCH = 16384

_info = plsc.get_sparse_core_info()
NC, NS, L = _info.num_cores, _info.num_subcores, _info.num_lanes
NW = NC * NS
BPW = BATCH // NW
CHUNK = 128


def _dot_body(w_i_hbm, w_j_hbm, wp_hbm, cp_hbm, partials_hbm,
              idx_i_v, idx_j_v, row_i_v, row_j_v, rows_i_v, rows_j_v,
              acc_v, sem_i, sem_j):
    wid = lax.axis_index("s") * NC + lax.axis_index("c")
    base = wid * BPW

    pltpu.sync_copy(w_i_hbm.at[pl.ds(base, BPW)], idx_i_v)
    pltpu.sync_copy(w_j_hbm.at[pl.ds(base, BPW)], idx_j_v)

    def to_rows(k, _):
        s = pl.ds(k * L, L)
        iv = idx_i_v[s]
        jv = idx_j_v[s]
        row_i_v[s] = ((iv >> 12) << 11) | (iv & 2047)
        row_j_v[s] = ((jv >> 12) << 11) | (jv & 2047)
        return 0

    lax.fori_loop(0, BPW // L, to_rows, 0, unroll=4)

    zero = jnp.zeros((L,), jnp.float32)
    accs = (zero, zero, zero, zero)
    for chunk in range(BPW // CHUNK):
        cb = chunk * CHUNK
        cp_i = pltpu.async_copy(
            wp_hbm.at[row_i_v.at[pl.ds(cb, CHUNK)]], rows_i_v, sem_i)
        cp_j = pltpu.async_copy(
            cp_hbm.at[row_j_v.at[pl.ds(cb, CHUNK)]], rows_j_v, sem_j)
        cp_i.wait()
        cp_j.wait()

        def dot_group(g, accs):
            a0, a1, a2, a3 = accs
            iv = idx_i_v[pl.ds(cb + g * L, L)]
            jv = idx_j_v[pl.ds(cb + g * L, L)]
            for t in range(L):
                k = g * L + t
                oi = ((iv[t] >> 11) & 1) * DIM
                oj = ((jv[t] >> 11) & 1) * DIM
                a0 = a0 + rows_i_v[k, pl.ds(oi, L)] * rows_j_v[k, pl.ds(oj, L)]
                a1 = a1 + (rows_i_v[k, pl.ds(oi + L, L)]
                           * rows_j_v[k, pl.ds(oj + L, L)])
                a2 = a2 + (rows_i_v[k, pl.ds(oi + 2 * L, L)]
                           * rows_j_v[k, pl.ds(oj + 2 * L, L)])
                a3 = a3 + (rows_i_v[k, pl.ds(oi + 3 * L, L)]
                           * rows_j_v[k, pl.ds(oj + 3 * L, L)])
            return (a0, a1, a2, a3)

        accs = lax.fori_loop(0, CHUNK // L, dot_group, accs)

    a0, a1, a2, a3 = accs
    acc_v[pl.ds(0, L)] = a0
    acc_v[pl.ds(L, L)] = a1
    acc_v[pl.ds(2 * L, L)] = a2
    acc_v[pl.ds(3 * L, L)] = a3
    for z in range(4, 8):
        acc_v[pl.ds(z * L, L)] = zero
    pltpu.sync_copy(acc_v, partials_hbm.at[pl.ds(wid * 128, 128)])


def _sc_dot(w_i, w_j, wp, cp):
    mesh = plsc.VectorSubcoreMesh(core_axis_name="c", subcore_axis_name="s")
    f = pl.kernel(
        _dot_body,
        out_type=jax.ShapeDtypeStruct((NW * 128,), jnp.float32),
        mesh=mesh,
        scratch_types=[
            pltpu.VMEM((BPW,), jnp.int32),
            pltpu.VMEM((BPW,), jnp.int32),
            pltpu.VMEM((BPW,), jnp.int32),
            pltpu.VMEM((BPW,), jnp.int32),
            pltpu.VMEM((CHUNK, 2 * DIM), jnp.float32),
            pltpu.VMEM((CHUNK, 2 * DIM), jnp.float32),
            pltpu.VMEM((128,), jnp.float32),
            pltpu.SemaphoreType.DMA,
            pltpu.SemaphoreType.DMA,
        ],
    )
    return f(w_i, w_j, wp, cp)


def _bias_body(w_i_hbm, w_j_hbm, w_bias_hbm, c_bias_hbm, bias_hbm,
               idx_i_v, idx_j_v, bi_v, bj_v, sem_bi, sem_bj):
    wid = lax.axis_index("s") * NC + lax.axis_index("c")
    base = wid * BPW

    pltpu.sync_copy(w_i_hbm.at[pl.ds(base, BPW)], idx_i_v)
    pltpu.sync_copy(w_j_hbm.at[pl.ds(base, BPW)], idx_j_v)

    cp_bi = pltpu.async_copy(w_bias_hbm.at[idx_i_v], bi_v, sem_bi)
    cp_bj = pltpu.async_copy(c_bias_hbm.at[idx_j_v], bj_v, sem_bj)
    cp_bi.wait()
    cp_bj.wait()

    def bias_step(k, _):
        s = pl.ds(k * L, L)
        bi_v[s] = bi_v[s] + bj_v[s]
        return 0

    lax.fori_loop(0, BPW // L, bias_step, 0, unroll=4)
    pltpu.sync_copy(bi_v, bias_hbm.at[pl.ds(base, BPW)])


def _sc_bias(w_i, w_j, w_bias, c_bias):
    mesh = plsc.VectorSubcoreMesh(core_axis_name="c", subcore_axis_name="s")
    f = pl.kernel(
        _bias_body,
        out_type=jax.ShapeDtypeStruct((BATCH,), jnp.float32),
        mesh=mesh,
        compiler_params=pltpu.CompilerParams(use_tc_tiling_on_sc=False),
        scratch_types=[
            pltpu.VMEM((BPW,), jnp.int32),
            pltpu.VMEM((BPW,), jnp.int32),
            pltpu.VMEM((BPW,), jnp.float32),
            pltpu.VMEM((BPW,), jnp.float32),
            pltpu.SemaphoreType.DMA,
            pltpu.SemaphoreType.DMA,
        ],
    )
    return f(w_i, w_j, w_bias, c_bias)


_PACK_W = 4096
_PACK_H = _PACK_W // 2
_PACK_GRID = (VOCAB + _PACK_W - 1) // _PACK_W
_PACK_ROWS = _PACK_GRID * _PACK_H


def _pack_body(wt_ref, ct_ref, ow_ref, oc_ref):
    w = wt_ref[...]
    ow_ref[...] = jnp.concatenate([w[:, :_PACK_H], w[:, _PACK_H:]], axis=0).T
    c = ct_ref[...]
    oc_ref[...] = jnp.concatenate([c[:, :_PACK_H], c[:, _PACK_H:]], axis=0).T


def _tc_pack(wt, ct):
    return pl.pallas_call(
        _pack_body,
        grid=(_PACK_GRID,),
        in_specs=[
            pl.BlockSpec((DIM, _PACK_W), lambda g: (0, g)),
            pl.BlockSpec((DIM, _PACK_W), lambda g: (0, g)),
        ],
        out_specs=[
            pl.BlockSpec((_PACK_H, 128), lambda g: (g, 0)),
            pl.BlockSpec((_PACK_H, 128), lambda g: (g, 0)),
        ],
        out_shape=(
            jax.ShapeDtypeStruct((_PACK_ROWS, 128), jnp.float32),
            jax.ShapeDtypeStruct((_PACK_ROWS, 128), jnp.float32),
        ),
    )(wt, ct)


def _tc_body(partials_ref, bias_ref, x_ref, loss_ref):
    x = jnp.sum(partials_ref[...])
    b = bias_ref[...]
    y_true = jnp.abs(b) + 1e-6
    t = jnp.abs(x) / 100.0 + jnp.zeros_like(b)
    weight = jnp.exp(0.75 * jnp.log(t))
    loss_ref[...] = weight * jnp.square(x - jnp.log(y_true))
    x_ref[...] = jnp.broadcast_to(x, (1, 1))


def _tc_loss(partials, bias2d):
    return pl.pallas_call(
        _tc_body,
        out_shape=(
            jax.ShapeDtypeStruct((1, 1), jnp.float32),
            jax.ShapeDtypeStruct(bias2d.shape, jnp.float32),
        ),
    )(partials, bias2d)


def kernel(w_i, w_j, w_emb, c_emb, w_bias, c_bias):
    w_i = w_i.astype(jnp.int32)
    w_j = w_j.astype(jnp.int32)
    wp, cp = _tc_pack(w_emb.T, c_emb.T)
    partials = _sc_dot(w_i, w_j, wp, cp)
    bias = _sc_bias(w_i, w_j, w_bias, c_bias)
    x, loss = _tc_loss(partials.reshape(NW, 128), bias.reshape(128, 128))
    return (x.reshape(()), loss.reshape(BATCH))


# speedup vs baseline: 6.5618x; 1.1635x over previous
import jax
import jax.numpy as jnp
from jax import lax
from jax.experimental import pallas as pl
from jax.experimental.pallas import tpu as pltpu
from jax.experimental.pallas import tpu_sc as plsc

VOCAB = 1000000
DIM = 64
BATCH = 16384

_info = plsc.get_sparse_core_info()
NC, NS, L = _info.num_cores, _info.num_subcores, _info.num_lanes
NW = NC * NS
BPW = BATCH // NW
CHUNK = 128


def _dot_body(w_i_hbm, w_j_hbm, wp_hbm, cp_hbm, partials_hbm,
              idx_i_v, idx_j_v, row_i_v, row_j_v, rows_i_v, rows_j_v,
              acc_v, sem_i, sem_j):
    wid = lax.axis_index("s") * NC + lax.axis_index("c")
    base = wid * BPW

    pltpu.sync_copy(w_i_hbm.at[pl.ds(base, BPW)], idx_i_v)
    pltpu.sync_copy(w_j_hbm.at[pl.ds(base, BPW)], idx_j_v)

    def to_rows(k, _):
        s = pl.ds(k * L, L)
        iv = idx_i_v[s]
        jv = idx_j_v[s]
        row_i_v[s] = ((iv >> 13) << 12) | (iv & 4095)
        row_j_v[s] = ((jv >> 13) << 12) | (jv & 4095)
        return 0

    lax.fori_loop(0, BPW // L, to_rows, 0, unroll=4)

    zero = jnp.zeros((L,), jnp.float32)
    accs = (zero, zero, zero, zero)
    for chunk in range(BPW // CHUNK):
        cb = chunk * CHUNK
        cp_i = pltpu.async_copy(
            wp_hbm.at[row_i_v.at[pl.ds(cb, CHUNK)]], rows_i_v, sem_i)
        cp_j = pltpu.async_copy(
            cp_hbm.at[row_j_v.at[pl.ds(cb, CHUNK)]], rows_j_v, sem_j)
        cp_i.wait()
        cp_j.wait()

        def dot_group(g, accs):
            a0, a1, a2, a3 = accs
            iv = idx_i_v[pl.ds(cb + g * L, L)]
            jv = idx_j_v[pl.ds(cb + g * L, L)]
            for t in range(L):
                k = g * L + t
                oi = ((iv[t] >> 12) & 1) * DIM
                oj = ((jv[t] >> 12) & 1) * DIM
                a0 = a0 + rows_i_v[k, pl.ds(oi, L)] * rows_j_v[k, pl.ds(oj, L)]
                a1 = a1 + (rows_i_v[k, pl.ds(oi + L, L)]
                           * rows_j_v[k, pl.ds(oj + L, L)])
                a2 = a2 + (rows_i_v[k, pl.ds(oi + 2 * L, L)]
                           * rows_j_v[k, pl.ds(oj + 2 * L, L)])
                a3 = a3 + (rows_i_v[k, pl.ds(oi + 3 * L, L)]
                           * rows_j_v[k, pl.ds(oj + 3 * L, L)])
            return (a0, a1, a2, a3)

        accs = lax.fori_loop(0, CHUNK // L, dot_group, accs)

    a0, a1, a2, a3 = accs
    acc_v[pl.ds(0, L)] = a0
    acc_v[pl.ds(L, L)] = a1
    acc_v[pl.ds(2 * L, L)] = a2
    acc_v[pl.ds(3 * L, L)] = a3
    for z in range(4, 8):
        acc_v[pl.ds(z * L, L)] = zero
    pltpu.sync_copy(acc_v, partials_hbm.at[pl.ds(wid * 128, 128)])


def _sc_dot(w_i, w_j, wp, cp):
    mesh = plsc.VectorSubcoreMesh(core_axis_name="c", subcore_axis_name="s")
    f = pl.kernel(
        _dot_body,
        out_type=jax.ShapeDtypeStruct((NW * 128,), jnp.float32),
        mesh=mesh,
        scratch_types=[
            pltpu.VMEM((BPW,), jnp.int32),
            pltpu.VMEM((BPW,), jnp.int32),
            pltpu.VMEM((BPW,), jnp.int32),
            pltpu.VMEM((BPW,), jnp.int32),
            pltpu.VMEM((CHUNK, 2 * DIM), jnp.float32),
            pltpu.VMEM((CHUNK, 2 * DIM), jnp.float32),
            pltpu.VMEM((128,), jnp.float32),
            pltpu.SemaphoreType.DMA,
            pltpu.SemaphoreType.DMA,
        ],
    )
    return f(w_i, w_j, wp, cp)


def _bias_body(w_i_hbm, w_j_hbm, w_bias_hbm, c_bias_hbm, bias_hbm,
               idx_i_v, idx_j_v, bi_v, bj_v, sem_bi, sem_bj):
    wid = lax.axis_index("s") * NC + lax.axis_index("c")
    base = wid * BPW

    pltpu.sync_copy(w_i_hbm.at[pl.ds(base, BPW)], idx_i_v)
    pltpu.sync_copy(w_j_hbm.at[pl.ds(base, BPW)], idx_j_v)

    cp_bi = pltpu.async_copy(w_bias_hbm.at[idx_i_v], bi_v, sem_bi)
    cp_bj = pltpu.async_copy(c_bias_hbm.at[idx_j_v], bj_v, sem_bj)
    cp_bi.wait()
    cp_bj.wait()

    def bias_step(k, _):
        s = pl.ds(k * L, L)
        bi_v[s] = bi_v[s] + bj_v[s]
        return 0

    lax.fori_loop(0, BPW // L, bias_step, 0, unroll=4)
    pltpu.sync_copy(bi_v, bias_hbm.at[pl.ds(base, BPW)])


def _sc_bias(w_i, w_j, w_bias, c_bias):
    mesh = plsc.VectorSubcoreMesh(core_axis_name="c", subcore_axis_name="s")
    f = pl.kernel(
        _bias_body,
        out_type=jax.ShapeDtypeStruct((BATCH,), jnp.float32),
        mesh=mesh,
        compiler_params=pltpu.CompilerParams(use_tc_tiling_on_sc=False),
        scratch_types=[
            pltpu.VMEM((BPW,), jnp.int32),
            pltpu.VMEM((BPW,), jnp.int32),
            pltpu.VMEM((BPW,), jnp.float32),
            pltpu.VMEM((BPW,), jnp.float32),
            pltpu.SemaphoreType.DMA,
            pltpu.SemaphoreType.DMA,
        ],
    )
    return f(w_i, w_j, w_bias, c_bias)


_PACK_W = 8192
_PACK_H = _PACK_W // 2
_PACK_GRID = (VOCAB + _PACK_W - 1) // _PACK_W
_PACK_ROWS = _PACK_GRID * _PACK_H


def _pack_body(wt_ref, ct_ref, ow_ref, oc_ref):
    w = wt_ref[...]
    ow_ref[...] = jnp.concatenate([w[:, :_PACK_H], w[:, _PACK_H:]], axis=0).T
    c = ct_ref[...]
    oc_ref[...] = jnp.concatenate([c[:, :_PACK_H], c[:, _PACK_H:]], axis=0).T


def _tc_pack(wt, ct):
    return pl.pallas_call(
        _pack_body,
        grid=(_PACK_GRID,),
        in_specs=[
            pl.BlockSpec((DIM, _PACK_W), lambda g: (0, g)),
            pl.BlockSpec((DIM, _PACK_W), lambda g: (0, g)),
        ],
        out_specs=[
            pl.BlockSpec((_PACK_H, 128), lambda g: (g, 0)),
            pl.BlockSpec((_PACK_H, 128), lambda g: (g, 0)),
        ],
        out_shape=(
            jax.ShapeDtypeStruct((_PACK_ROWS, 128), jnp.float32),
            jax.ShapeDtypeStruct((_PACK_ROWS, 128), jnp.float32),
        ),
    )(wt, ct)


def _tc_body(partials_ref, bias_ref, x_ref, loss_ref):
    x = jnp.sum(partials_ref[...])
    b = bias_ref[...]
    y_true = jnp.abs(b) + 1e-6
    t = jnp.abs(x) / 100.0 + jnp.zeros_like(b)
    weight = jnp.exp(0.75 * jnp.log(t))
    loss_ref[...] = weight * jnp.square(x - jnp.log(y_true))
    x_ref[...] = jnp.broadcast_to(x, (1, 1))


def _tc_loss(partials, bias2d):
    return pl.pallas_call(
        _tc_body,
        out_shape=(
            jax.ShapeDtypeStruct((1, 1), jnp.float32),
            jax.ShapeDtypeStruct(bias2d.shape, jnp.float32),
        ),
    )(partials, bias2d)


def kernel(w_i, w_j, w_emb, c_emb, w_bias, c_bias):
    w_i = w_i.astype(jnp.int32)
    w_j = w_j.astype(jnp.int32)
    wp, cp = _tc_pack(w_emb.T, c_emb.T)
    partials = _sc_dot(w_i, w_j, wp, cp)
    bias = _sc_bias(w_i, w_j, w_bias, c_bias)
    x, loss = _tc_loss(partials.reshape(NW, 128), bias.reshape(128, 128))
    return (x.reshape(()), loss.reshape(BATCH))


# speedup vs baseline: 6.6551x; 1.0142x over previous
import jax
import jax.numpy as jnp
from jax import lax
from jax.experimental import pallas as pl
from jax.experimental.pallas import tpu as pltpu
from jax.experimental.pallas import tpu_sc as plsc

VOCAB = 1000000
DIM = 64
BATCH = 16384

_info = plsc.get_sparse_core_info()
NC, NS, L = _info.num_cores, _info.num_subcores, _info.num_lanes
NW = NC * NS
BPW = BATCH // NW
CHUNK = 128


def _dot_body(w_i_hbm, w_j_hbm, wp_hbm, cp_hbm, partials_hbm,
              idx_i_v, idx_j_v, row_i_v, row_j_v, rows_i_v, rows_j_v,
              acc_v, sem_i, sem_j):
    wid = lax.axis_index("s") * NC + lax.axis_index("c")
    base = wid * BPW

    pltpu.sync_copy(w_i_hbm.at[pl.ds(base, BPW)], idx_i_v)
    pltpu.sync_copy(w_j_hbm.at[pl.ds(base, BPW)], idx_j_v)

    def to_rows(k, _):
        s = pl.ds(k * L, L)
        iv = idx_i_v[s]
        jv = idx_j_v[s]
        row_i_v[s] = ((iv >> 14) << 13) | (iv & 8191)
        row_j_v[s] = ((jv >> 14) << 13) | (jv & 8191)
        return 0

    lax.fori_loop(0, BPW // L, to_rows, 0, unroll=4)

    zero = jnp.zeros((L,), jnp.float32)
    accs = (zero, zero, zero, zero)
    for chunk in range(BPW // CHUNK):
        cb = chunk * CHUNK
        cp_i = pltpu.async_copy(
            wp_hbm.at[row_i_v.at[pl.ds(cb, CHUNK)]], rows_i_v, sem_i)
        cp_j = pltpu.async_copy(
            cp_hbm.at[row_j_v.at[pl.ds(cb, CHUNK)]], rows_j_v, sem_j)
        cp_i.wait()
        cp_j.wait()

        def dot_group(g, accs):
            a0, a1, a2, a3 = accs
            iv = idx_i_v[pl.ds(cb + g * L, L)]
            jv = idx_j_v[pl.ds(cb + g * L, L)]
            for t in range(L):
                k = g * L + t
                oi = ((iv[t] >> 13) & 1) * DIM
                oj = ((jv[t] >> 13) & 1) * DIM
                a0 = a0 + rows_i_v[k, pl.ds(oi, L)] * rows_j_v[k, pl.ds(oj, L)]
                a1 = a1 + (rows_i_v[k, pl.ds(oi + L, L)]
                           * rows_j_v[k, pl.ds(oj + L, L)])
                a2 = a2 + (rows_i_v[k, pl.ds(oi + 2 * L, L)]
                           * rows_j_v[k, pl.ds(oj + 2 * L, L)])
                a3 = a3 + (rows_i_v[k, pl.ds(oi + 3 * L, L)]
                           * rows_j_v[k, pl.ds(oj + 3 * L, L)])
            return (a0, a1, a2, a3)

        accs = lax.fori_loop(0, CHUNK // L, dot_group, accs)

    a0, a1, a2, a3 = accs
    acc_v[pl.ds(0, L)] = a0
    acc_v[pl.ds(L, L)] = a1
    acc_v[pl.ds(2 * L, L)] = a2
    acc_v[pl.ds(3 * L, L)] = a3
    for z in range(4, 8):
        acc_v[pl.ds(z * L, L)] = zero
    pltpu.sync_copy(acc_v, partials_hbm.at[pl.ds(wid * 128, 128)])


def _sc_dot(w_i, w_j, wp, cp):
    mesh = plsc.VectorSubcoreMesh(core_axis_name="c", subcore_axis_name="s")
    f = pl.kernel(
        _dot_body,
        out_type=jax.ShapeDtypeStruct((NW * 128,), jnp.float32),
        mesh=mesh,
        scratch_types=[
            pltpu.VMEM((BPW,), jnp.int32),
            pltpu.VMEM((BPW,), jnp.int32),
            pltpu.VMEM((BPW,), jnp.int32),
            pltpu.VMEM((BPW,), jnp.int32),
            pltpu.VMEM((CHUNK, 2 * DIM), jnp.float32),
            pltpu.VMEM((CHUNK, 2 * DIM), jnp.float32),
            pltpu.VMEM((128,), jnp.float32),
            pltpu.SemaphoreType.DMA,
            pltpu.SemaphoreType.DMA,
        ],
    )
    return f(w_i, w_j, wp, cp)


def _bias_body(w_i_hbm, w_j_hbm, w_bias_hbm, c_bias_hbm, bias_hbm,
               idx_i_v, idx_j_v, bi_v, bj_v, sem_bi, sem_bj):
    wid = lax.axis_index("s") * NC + lax.axis_index("c")
    base = wid * BPW

    pltpu.sync_copy(w_i_hbm.at[pl.ds(base, BPW)], idx_i_v)
    pltpu.sync_copy(w_j_hbm.at[pl.ds(base, BPW)], idx_j_v)

    cp_bi = pltpu.async_copy(w_bias_hbm.at[idx_i_v], bi_v, sem_bi)
    cp_bj = pltpu.async_copy(c_bias_hbm.at[idx_j_v], bj_v, sem_bj)
    cp_bi.wait()
    cp_bj.wait()

    def bias_step(k, _):
        s = pl.ds(k * L, L)
        bi_v[s] = bi_v[s] + bj_v[s]
        return 0

    lax.fori_loop(0, BPW // L, bias_step, 0, unroll=4)
    pltpu.sync_copy(bi_v, bias_hbm.at[pl.ds(base, BPW)])


def _sc_bias(w_i, w_j, w_bias, c_bias):
    mesh = plsc.VectorSubcoreMesh(core_axis_name="c", subcore_axis_name="s")
    f = pl.kernel(
        _bias_body,
        out_type=jax.ShapeDtypeStruct((BATCH,), jnp.float32),
        mesh=mesh,
        compiler_params=pltpu.CompilerParams(use_tc_tiling_on_sc=False),
        scratch_types=[
            pltpu.VMEM((BPW,), jnp.int32),
            pltpu.VMEM((BPW,), jnp.int32),
            pltpu.VMEM((BPW,), jnp.float32),
            pltpu.VMEM((BPW,), jnp.float32),
            pltpu.SemaphoreType.DMA,
            pltpu.SemaphoreType.DMA,
        ],
    )
    return f(w_i, w_j, w_bias, c_bias)


_PACK_W = 16384
_PACK_H = _PACK_W // 2
_PACK_GRID = (VOCAB + _PACK_W - 1) // _PACK_W
_PACK_ROWS = _PACK_GRID * _PACK_H


def _pack_body(wt_ref, ct_ref, ow_ref, oc_ref):
    w = wt_ref[...]
    ow_ref[...] = jnp.concatenate([w[:, :_PACK_H], w[:, _PACK_H:]], axis=0).T
    c = ct_ref[...]
    oc_ref[...] = jnp.concatenate([c[:, :_PACK_H], c[:, _PACK_H:]], axis=0).T


def _tc_pack(wt, ct):
    return pl.pallas_call(
        _pack_body,
        grid=(_PACK_GRID,),
        in_specs=[
            pl.BlockSpec((DIM, _PACK_W), lambda g: (0, g)),
            pl.BlockSpec((DIM, _PACK_W), lambda g: (0, g)),
        ],
        out_specs=[
            pl.BlockSpec((_PACK_H, 128), lambda g: (g, 0)),
            pl.BlockSpec((_PACK_H, 128), lambda g: (g, 0)),
        ],
        out_shape=(
            jax.ShapeDtypeStruct((_PACK_ROWS, 128), jnp.float32),
            jax.ShapeDtypeStruct((_PACK_ROWS, 128), jnp.float32),
        ),
    )(wt, ct)


def _tc_body(partials_ref, bias_ref, x_ref, loss_ref):
    x = jnp.sum(partials_ref[...])
    b = bias_ref[...]
    y_true = jnp.abs(b) + 1e-6
    t = jnp.abs(x) / 100.0 + jnp.zeros_like(b)
    weight = jnp.exp(0.75 * jnp.log(t))
    loss_ref[...] = weight * jnp.square(x - jnp.log(y_true))
    x_ref[...] = jnp.broadcast_to(x, (1, 1))


def _tc_loss(partials, bias2d):
    return pl.pallas_call(
        _tc_body,
        out_shape=(
            jax.ShapeDtypeStruct((1, 1), jnp.float32),
            jax.ShapeDtypeStruct(bias2d.shape, jnp.float32),
        ),
    )(partials, bias2d)


def kernel(w_i, w_j, w_emb, c_emb, w_bias, c_bias):
    w_i = w_i.astype(jnp.int32)
    w_j = w_j.astype(jnp.int32)
    wp, cp = _tc_pack(w_emb.T, c_emb.T)
    partials = _sc_dot(w_i, w_j, wp, cp)
    bias = _sc_bias(w_i, w_j, w_bias, c_bias)
    x, loss = _tc_loss(partials.reshape(NW, 128), bias.reshape(128, 128))
    return (x.reshape(()), loss.reshape(BATCH))


# speedup vs baseline: 6.7165x; 1.0092x over previous
import jax
import jax.numpy as jnp
from jax import lax
from jax.experimental import pallas as pl
from jax.experimental.pallas import tpu as pltpu
from jax.experimental.pallas import tpu_sc as plsc

VOCAB = 1000000
DIM = 64
BATCH = 16384

_info = plsc.get_sparse_core_info()
NC, NS, L = _info.num_cores, _info.num_subcores, _info.num_lanes
NW = NC * NS
BPW = BATCH // NW
CHUNK = 128


def _dot_body(w_i_hbm, w_j_hbm, wp_hbm, cp_hbm, partials_hbm,
              idx_i_v, idx_j_v, row_i_v, row_j_v,
              rows_i0, rows_j0, rows_i1, rows_j1,
              acc_v, sem_i0, sem_j0, sem_i1, sem_j1):
    wid = lax.axis_index("s") * NC + lax.axis_index("c")
    base = wid * BPW

    pltpu.sync_copy(w_i_hbm.at[pl.ds(base, BPW)], idx_i_v)
    pltpu.sync_copy(w_j_hbm.at[pl.ds(base, BPW)], idx_j_v)

    def to_rows(k, _):
        s = pl.ds(k * L, L)
        iv = idx_i_v[s]
        jv = idx_j_v[s]
        row_i_v[s] = ((iv >> 14) << 13) | (iv & 8191)
        row_j_v[s] = ((jv >> 14) << 13) | (jv & 8191)
        return 0

    lax.fori_loop(0, BPW // L, to_rows, 0, unroll=4)

    bufs = ((rows_i0, rows_j0, sem_i0, sem_j0),
            (rows_i1, rows_j1, sem_i1, sem_j1))

    def issue(chunk):
        bi, bj, si, sj = bufs[chunk % 2]
        cb = chunk * CHUNK
        ci = pltpu.async_copy(wp_hbm.at[row_i_v.at[pl.ds(cb, CHUNK)]], bi, si)
        cj = pltpu.async_copy(cp_hbm.at[row_j_v.at[pl.ds(cb, CHUNK)]], bj, sj)
        return ci, cj

    zero = jnp.zeros((L,), jnp.float32)
    accs = (zero, zero, zero, zero)
    nchunks = BPW // CHUNK
    pend = issue(0)
    for chunk in range(nchunks):
        cb = chunk * CHUNK
        bi, bj, _, _ = bufs[chunk % 2]
        pend[0].wait()
        pend[1].wait()
        if chunk + 1 < nchunks:
            pend = issue(chunk + 1)

        def dot_group(g, accs, bi=bi, bj=bj, cb=cb):
            a0, a1, a2, a3 = accs
            iv = idx_i_v[pl.ds(cb + g * L, L)]
            jv = idx_j_v[pl.ds(cb + g * L, L)]
            for t in range(L):
                k = g * L + t
                oi = ((iv[t] >> 13) & 1) * DIM
                oj = ((jv[t] >> 13) & 1) * DIM
                a0 = a0 + bi[k, pl.ds(oi, L)] * bj[k, pl.ds(oj, L)]
                a1 = a1 + bi[k, pl.ds(oi + L, L)] * bj[k, pl.ds(oj + L, L)]
                a2 = a2 + (bi[k, pl.ds(oi + 2 * L, L)]
                           * bj[k, pl.ds(oj + 2 * L, L)])
                a3 = a3 + (bi[k, pl.ds(oi + 3 * L, L)]
                           * bj[k, pl.ds(oj + 3 * L, L)])
            return (a0, a1, a2, a3)

        accs = lax.fori_loop(0, CHUNK // L, dot_group, accs)

    a0, a1, a2, a3 = accs
    acc_v[pl.ds(0, L)] = a0
    acc_v[pl.ds(L, L)] = a1
    acc_v[pl.ds(2 * L, L)] = a2
    acc_v[pl.ds(3 * L, L)] = a3
    for z in range(4, 8):
        acc_v[pl.ds(z * L, L)] = zero
    pltpu.sync_copy(acc_v, partials_hbm.at[pl.ds(wid * 128, 128)])


def _sc_dot(w_i, w_j, wp, cp):
    mesh = plsc.VectorSubcoreMesh(core_axis_name="c", subcore_axis_name="s")
    f = pl.kernel(
        _dot_body,
        out_type=jax.ShapeDtypeStruct((NW * 128,), jnp.float32),
        mesh=mesh,
        scratch_types=[
            pltpu.VMEM((BPW,), jnp.int32),
            pltpu.VMEM((BPW,), jnp.int32),
            pltpu.VMEM((BPW,), jnp.int32),
            pltpu.VMEM((BPW,), jnp.int32),
            pltpu.VMEM((CHUNK, 2 * DIM), jnp.float32),
            pltpu.VMEM((CHUNK, 2 * DIM), jnp.float32),
            pltpu.VMEM((CHUNK, 2 * DIM), jnp.float32),
            pltpu.VMEM((CHUNK, 2 * DIM), jnp.float32),
            pltpu.VMEM((128,), jnp.float32),
            pltpu.SemaphoreType.DMA,
            pltpu.SemaphoreType.DMA,
            pltpu.SemaphoreType.DMA,
            pltpu.SemaphoreType.DMA,
        ],
    )
    return f(w_i, w_j, wp, cp)


def _bias_body(w_i_hbm, w_j_hbm, w_bias_hbm, c_bias_hbm, bias_hbm,
               idx_i_v, idx_j_v, bi_v, bj_v, sem_bi, sem_bj):
    wid = lax.axis_index("s") * NC + lax.axis_index("c")
    base = wid * BPW

    pltpu.sync_copy(w_i_hbm.at[pl.ds(base, BPW)], idx_i_v)
    pltpu.sync_copy(w_j_hbm.at[pl.ds(base, BPW)], idx_j_v)

    cp_bi = pltpu.async_copy(w_bias_hbm.at[idx_i_v], bi_v, sem_bi)
    cp_bj = pltpu.async_copy(c_bias_hbm.at[idx_j_v], bj_v, sem_bj)
    cp_bi.wait()
    cp_bj.wait()

    def bias_step(k, _):
        s = pl.ds(k * L, L)
        bi_v[s] = bi_v[s] + bj_v[s]
        return 0

    lax.fori_loop(0, BPW // L, bias_step, 0, unroll=4)
    pltpu.sync_copy(bi_v, bias_hbm.at[pl.ds(base, BPW)])


def _sc_bias(w_i, w_j, w_bias, c_bias):
    mesh = plsc.VectorSubcoreMesh(core_axis_name="c", subcore_axis_name="s")
    f = pl.kernel(
        _bias_body,
        out_type=jax.ShapeDtypeStruct((BATCH,), jnp.float32),
        mesh=mesh,
        compiler_params=pltpu.CompilerParams(use_tc_tiling_on_sc=False),
        scratch_types=[
            pltpu.VMEM((BPW,), jnp.int32),
            pltpu.VMEM((BPW,), jnp.int32),
            pltpu.VMEM((BPW,), jnp.float32),
            pltpu.VMEM((BPW,), jnp.float32),
            pltpu.SemaphoreType.DMA,
            pltpu.SemaphoreType.DMA,
        ],
    )
    return f(w_i, w_j, w_bias, c_bias)


_PACK_W = 16384
_PACK_H = _PACK_W // 2
_PACK_GRID = (VOCAB + _PACK_W - 1) // _PACK_W
_PACK_ROWS = _PACK_GRID * _PACK_H


def _pack_body(wt_ref, ct_ref, ow_ref, oc_ref):
    w = wt_ref[...]
    ow_ref[...] = jnp.concatenate([w[:, :_PACK_H], w[:, _PACK_H:]], axis=0).T
    c = ct_ref[...]
    oc_ref[...] = jnp.concatenate([c[:, :_PACK_H], c[:, _PACK_H:]], axis=0).T


def _tc_pack(wt, ct):
    return pl.pallas_call(
        _pack_body,
        grid=(_PACK_GRID,),
        in_specs=[
            pl.BlockSpec((DIM, _PACK_W), lambda g: (0, g)),
            pl.BlockSpec((DIM, _PACK_W), lambda g: (0, g)),
        ],
        out_specs=[
            pl.BlockSpec((_PACK_H, 128), lambda g: (g, 0)),
            pl.BlockSpec((_PACK_H, 128), lambda g: (g, 0)),
        ],
        out_shape=(
            jax.ShapeDtypeStruct((_PACK_ROWS, 128), jnp.float32),
            jax.ShapeDtypeStruct((_PACK_ROWS, 128), jnp.float32),
        ),
    )(wt, ct)


def _tc_body(partials_ref, bias_ref, x_ref, loss_ref):
    x = jnp.sum(partials_ref[...])
    b = bias_ref[...]
    y_true = jnp.abs(b) + 1e-6
    t = jnp.abs(x) / 100.0 + jnp.zeros_like(b)
    weight = jnp.exp(0.75 * jnp.log(t))
    loss_ref[...] = weight * jnp.square(x - jnp.log(y_true))
    x_ref[...] = jnp.broadcast_to(x, (1, 1))


def _tc_loss(partials, bias2d):
    return pl.pallas_call(
        _tc_body,
        out_shape=(
            jax.ShapeDtypeStruct((1, 1), jnp.float32),
            jax.ShapeDtypeStruct(bias2d.shape, jnp.float32),
        ),
    )(partials, bias2d)


def kernel(w_i, w_j, w_emb, c_emb, w_bias, c_bias):
    w_i = w_i.astype(jnp.int32)
    w_j = w_j.astype(jnp.int32)
    wp, cp = _tc_pack(w_emb.T, c_emb.T)
    partials = _sc_dot(w_i, w_j, wp, cp)
    bias = _sc_bias(w_i, w_j, w_bias, c_bias)
    x, loss = _tc_loss(partials.reshape(NW, 128), bias.reshape(128, 128))
    return (x.reshape(()), loss.reshape(BATCH))


# speedup vs baseline: 6.7170x; 1.0001x over previous
import jax
import jax.numpy as jnp
from jax import lax
from jax.experimental import pallas as pl
from jax.experimental.pallas import tpu as pltpu
from jax.experimental.pallas import tpu_sc as plsc

VOCAB = 1000000
DIM = 64
BATCH = 16384

_info = plsc.get_sparse_core_info()
NC, NS, L = _info.num_cores, _info.num_subcores, _info.num_lanes
NW = NC * NS
BPW = BATCH // NW
CHUNK = 128


def _dot_body(w_i_hbm, w_j_hbm, wp_hbm, cp_hbm, partials_hbm,
              idx_i_v, idx_j_v, row_i_v, row_j_v,
              rows_i0, rows_j0, rows_i1, rows_j1,
              acc_v, sem_i0, sem_j0, sem_i1, sem_j1):
    wid = lax.axis_index("s") * NC + lax.axis_index("c")
    base = wid * BPW

    pltpu.sync_copy(w_i_hbm.at[pl.ds(base, BPW)], idx_i_v)
    pltpu.sync_copy(w_j_hbm.at[pl.ds(base, BPW)], idx_j_v)

    def to_rows(k, _):
        s = pl.ds(k * L, L)
        iv = idx_i_v[s]
        jv = idx_j_v[s]
        row_i_v[s] = ((iv >> 14) << 13) | (iv & 8191)
        row_j_v[s] = ((jv >> 14) << 13) | (jv & 8191)
        return 0

    lax.fori_loop(0, BPW // L, to_rows, 0, unroll=4)

    bufs = ((rows_i0, rows_j0, sem_i0, sem_j0),
            (rows_i1, rows_j1, sem_i1, sem_j1))

    def issue(chunk):
        bi, bj, si, sj = bufs[chunk % 2]
        cb = chunk * CHUNK
        ci = pltpu.async_copy(wp_hbm.at[row_i_v.at[pl.ds(cb, CHUNK)]], bi, si)
        cj = pltpu.async_copy(cp_hbm.at[row_j_v.at[pl.ds(cb, CHUNK)]], bj, sj)
        return ci, cj

    zero = jnp.zeros((L,), jnp.float32)
    accs = (zero, zero, zero, zero)
    nchunks = BPW // CHUNK
    pend = issue(0)
    for chunk in range(nchunks):
        cb = chunk * CHUNK
        bi, bj, _, _ = bufs[chunk % 2]
        pend[0].wait()
        pend[1].wait()
        if chunk + 1 < nchunks:
            pend = issue(chunk + 1)

        def dot_group(g, accs, bi=bi, bj=bj, cb=cb):
            a0, a1, a2, a3 = accs
            iv = idx_i_v[pl.ds(cb + g * L, L)]
            jv = idx_j_v[pl.ds(cb + g * L, L)]
            for t in range(L):
                k = g * L + t
                oi = ((iv[t] >> 13) & 1) * DIM
                oj = ((jv[t] >> 13) & 1) * DIM
                a0 = a0 + bi[k, pl.ds(oi, L)] * bj[k, pl.ds(oj, L)]
                a1 = a1 + bi[k, pl.ds(oi + L, L)] * bj[k, pl.ds(oj + L, L)]
                a2 = a2 + (bi[k, pl.ds(oi + 2 * L, L)]
                           * bj[k, pl.ds(oj + 2 * L, L)])
                a3 = a3 + (bi[k, pl.ds(oi + 3 * L, L)]
                           * bj[k, pl.ds(oj + 3 * L, L)])
            return (a0, a1, a2, a3)

        accs = lax.fori_loop(0, CHUNK // L, dot_group, accs)

    a0, a1, a2, a3 = accs
    acc_v[pl.ds(0, L)] = a0
    acc_v[pl.ds(L, L)] = a1
    acc_v[pl.ds(2 * L, L)] = a2
    acc_v[pl.ds(3 * L, L)] = a3
    for z in range(4, 8):
        acc_v[pl.ds(z * L, L)] = zero
    pltpu.sync_copy(acc_v, partials_hbm.at[pl.ds(wid * 128, 128)])


def _sc_dot(w_i, w_j, wp, cp):
    mesh = plsc.VectorSubcoreMesh(core_axis_name="c", subcore_axis_name="s")
    f = pl.kernel(
        _dot_body,
        out_type=jax.ShapeDtypeStruct((NW * 128,), jnp.float32),
        mesh=mesh,
        scratch_types=[
            pltpu.VMEM((BPW,), jnp.int32),
            pltpu.VMEM((BPW,), jnp.int32),
            pltpu.VMEM((BPW,), jnp.int32),
            pltpu.VMEM((BPW,), jnp.int32),
            pltpu.VMEM((CHUNK, 2 * DIM), jnp.float32),
            pltpu.VMEM((CHUNK, 2 * DIM), jnp.float32),
            pltpu.VMEM((CHUNK, 2 * DIM), jnp.float32),
            pltpu.VMEM((CHUNK, 2 * DIM), jnp.float32),
            pltpu.VMEM((128,), jnp.float32),
            pltpu.SemaphoreType.DMA,
            pltpu.SemaphoreType.DMA,
            pltpu.SemaphoreType.DMA,
            pltpu.SemaphoreType.DMA,
        ],
    )
    return f(w_i, w_j, wp, cp)


def _bias_body(w_i_hbm, w_j_hbm, w_bias_hbm, c_bias_hbm, bias_hbm,
               idx_i_v, idx_j_v, bi_v, bj_v, sem_bi, sem_bj):
    wid = lax.axis_index("s") * NC + lax.axis_index("c")
    base = wid * BPW

    pltpu.sync_copy(w_i_hbm.at[pl.ds(base, BPW)], idx_i_v)
    pltpu.sync_copy(w_j_hbm.at[pl.ds(base, BPW)], idx_j_v)

    cp_bi = pltpu.async_copy(w_bias_hbm.at[idx_i_v], bi_v, sem_bi)
    cp_bj = pltpu.async_copy(c_bias_hbm.at[idx_j_v], bj_v, sem_bj)
    cp_bi.wait()
    cp_bj.wait()

    def bias_step(k, _):
        s = pl.ds(k * L, L)
        bi_v[s] = bi_v[s] + bj_v[s]
        return 0

    lax.fori_loop(0, BPW // L, bias_step, 0, unroll=4)
    pltpu.sync_copy(bi_v, bias_hbm.at[pl.ds(base, BPW)])


def _sc_bias(w_i, w_j, w_bias, c_bias):
    mesh = plsc.VectorSubcoreMesh(core_axis_name="c", subcore_axis_name="s")
    f = pl.kernel(
        _bias_body,
        out_type=jax.ShapeDtypeStruct((BATCH,), jnp.float32),
        mesh=mesh,
        compiler_params=pltpu.CompilerParams(use_tc_tiling_on_sc=False),
        scratch_types=[
            pltpu.VMEM((BPW,), jnp.int32),
            pltpu.VMEM((BPW,), jnp.int32),
            pltpu.VMEM((BPW,), jnp.float32),
            pltpu.VMEM((BPW,), jnp.float32),
            pltpu.SemaphoreType.DMA,
            pltpu.SemaphoreType.DMA,
        ],
    )
    return f(w_i, w_j, w_bias, c_bias)


_PACK_W = 16384
_PACK_H = _PACK_W // 2
_PACK_GRID = (VOCAB + _PACK_W - 1) // _PACK_W
_PACK_ROWS = _PACK_GRID * _PACK_H


def _pack_body(wt_ref, ct_ref, ow_ref, oc_ref):
    w = wt_ref[...]
    ow_ref[...] = jnp.concatenate([w[:, :_PACK_H], w[:, _PACK_H:]], axis=0).T
    c = ct_ref[...]
    oc_ref[...] = jnp.concatenate([c[:, :_PACK_H], c[:, _PACK_H:]], axis=0).T


def _tc_pack(wt, ct):
    return pl.pallas_call(
        _pack_body,
        grid=(_PACK_GRID,),
        compiler_params=pltpu.CompilerParams(
            vmem_limit_bytes=100 * 1024 * 1024),
        in_specs=[
            pl.BlockSpec((DIM, _PACK_W), lambda g: (0, g)),
            pl.BlockSpec((DIM, _PACK_W), lambda g: (0, g)),
        ],
        out_specs=[
            pl.BlockSpec((_PACK_H, 128), lambda g: (g, 0)),
            pl.BlockSpec((_PACK_H, 128), lambda g: (g, 0)),
        ],
        out_shape=(
            jax.ShapeDtypeStruct((_PACK_ROWS, 128), jnp.float32),
            jax.ShapeDtypeStruct((_PACK_ROWS, 128), jnp.float32),
        ),
    )(wt, ct)


def _tc_body(partials_ref, bias_ref, x_ref, loss_ref):
    x = jnp.sum(partials_ref[...])
    b = bias_ref[...]
    y_true = jnp.abs(b) + 1e-6
    t = jnp.abs(x) / 100.0 + jnp.zeros_like(b)
    weight = jnp.exp(0.75 * jnp.log(t))
    loss_ref[...] = weight * jnp.square(x - jnp.log(y_true))
    x_ref[...] = jnp.broadcast_to(x, (1, 1))


def _tc_loss(partials, bias2d):
    return pl.pallas_call(
        _tc_body,
        out_shape=(
            jax.ShapeDtypeStruct((1, 1), jnp.float32),
            jax.ShapeDtypeStruct(bias2d.shape, jnp.float32),
        ),
    )(partials, bias2d)


def kernel(w_i, w_j, w_emb, c_emb, w_bias, c_bias):
    w_i = w_i.astype(jnp.int32)
    w_j = w_j.astype(jnp.int32)
    wp, cp = _tc_pack(w_emb.T, c_emb.T)
    partials = _sc_dot(w_i, w_j, wp, cp)
    bias = _sc_bias(w_i, w_j, w_bias, c_bias)
    x, loss = _tc_loss(partials.reshape(NW, 128), bias.reshape(128, 128))
    return (x.reshape(()), loss.reshape(BATCH))
